# baseline (speedup 1.0000x reference)
import functools

import jax
import jax.numpy as jnp
from jax import lax
from jax.experimental import pallas as pl
from jax.experimental.pallas import tpu as pltpu
from jax.experimental.pallas import tpu_sc as plsc

NC = 2
NS = 16
NW = NC * NS
C = 128
DP = 64
NV = DP // 16


def _proj_kernel(emb_ref, w_ref, out_ref):
    out_ref[...] = jnp.dot(emb_ref[...].astype(jnp.bfloat16),
                           w_ref[...].astype(jnp.bfloat16),
                           preferred_element_type=jnp.float32
                           ).astype(jnp.bfloat16)


def _proj(emb, W1):
    V, D = emb.shape
    Dout = W1.shape[1]
    BS = 10000
    return pl.pallas_call(
        _proj_kernel,
        grid=(V // BS,),
        in_specs=[
            pl.BlockSpec((BS, D), lambda i: (i, 0)),
            pl.BlockSpec((D, Dout), lambda i: (0, 0)),
        ],
        out_specs=pl.BlockSpec((BS, Dout), lambda i: (i, 0)),
        out_shape=jax.ShapeDtypeStruct((V, Dout), jnp.bfloat16),
    )(emb, W1)


def _scatter_add_rows(src_ref, acc_ref, seg_ref):
    pltpu.sync_copy(src_ref, acc_ref.at[seg_ref], add=True)


def _tile_slice(acc_sh, sid):
    return acc_sh.at[sid]


def _bag_body(tab, xt, xo, b1, out,
              xo_v, idx_a, idx_b, buf_a, buf_b, seg_a, seg_b, b1_v, acc_sh,
              out_v, inv_v, isem_a, isem_b, gsem_a, gsem_b):
    BPW = out_v.shape[0]
    sid = lax.axis_index("s")
    wid = lax.axis_index("c") * NS + sid
    acc = _tile_slice(acc_sh, sid)
    b0 = pl.multiple_of(wid * BPW, BPW)
    pltpu.sync_copy(xo.at[pl.ds(b0, BPW + 24)], xo_v)
    pltpu.sync_copy(b1, b1_v)
    hdr = xo_v[pl.ds(0, 16)]
    s0 = hdr[0]
    e0 = xo_v[pl.ds(BPW, 16)][0]
    base = pl.multiple_of(jnp.bitwise_and(s0, jnp.int32(-8)), 8)
    nchunks = jnp.maximum(lax.div(e0 - base + (C - 1), jnp.int32(C)),
                          jnp.int32(1))

    zf = jnp.zeros((16,), jnp.float32)
    zi = jnp.zeros((16,), jnp.int32)
    ones = jnp.ones((16,), jnp.int32)
    lane = lax.iota(jnp.int32, 16)
    dummy = jnp.full((16,), jnp.int32(BPW), jnp.int32)

    def zero_out(r, u):
        for k in range(NV):
            out_v[r, k * 16:(k + 1) * 16] = zf
        return u

    lax.fori_loop(jnp.int32(0), jnp.int32(BPW), zero_out, jnp.int32(0))
    pltpu.sync_copy(out_v, acc.at[pl.ds(0, BPW)])

    def idx_start(c, idx_v, isem):
        off = pl.multiple_of(base + c * C, 8)
        pltpu.async_copy(xt.at[pl.ds(off, C)], idx_v, isem)

    def idx_wait(c, idx_v, isem):
        off = pl.multiple_of(base + c * C, 8)
        pltpu.make_async_copy(xt.at[pl.ds(off, C)], idx_v, isem).wait()

    def g_start(idx_v, buf_v, gsem):
        pltpu.async_copy(tab.at[idx_v], buf_v, gsem)

    def g_wait(idx_v, buf_v, gsem):
        pltpu.make_async_copy(tab.at[idx_v], buf_v, gsem).wait()

    idx_start(jnp.int32(0), idx_a, isem_a)

    @pl.when(nchunks > 1)
    def _():
        idx_start(jnp.int32(1), idx_b, isem_b)

    idx_wait(jnp.int32(0), idx_a, isem_a)
    g_start(idx_a, buf_a, gsem_a)

    def half(c, idx_v, isem, buf_v, seg_v, n_idx_v, n_isem, n_buf_v, n_gsem,
             gsem, cur_b):
        @pl.when(c < nchunks)
        def _():
            g_wait(idx_v, buf_v, gsem)

        @pl.when(c + 2 < nchunks)
        def _():
            idx_start(c + 2, idx_v, isem)

        @pl.when(c + 1 < nchunks)
        def _():
            idx_wait(c + 1, n_idx_v, n_isem)
            g_start(n_idx_v, n_buf_v, n_gsem)

        lo = base + c * C
        hi = jnp.minimum(lo + C, e0)

        total = cur_b


        return total

    def pair_body(p, cur_b):
        c = p * 2
        cur_b = half(c, idx_a, isem_a, buf_a, seg_a, idx_b, isem_b, buf_b,
                     gsem_b, gsem_a, cur_b)
        cur_b = half(c + 1, idx_b, isem_b, buf_b, seg_b, idx_a, isem_a,
                     buf_a, gsem_a, gsem_b, cur_b)
        return cur_b

    npairs = lax.div(nchunks + 1, jnp.int32(2))
    lax.fori_loop(jnp.int32(0), npairs, pair_body, jnp.int32(0))

    def inv_grp(g, u):
        b = g * 16
        v0 = xo_v[pl.ds(b, 16)]
        v1 = xo_v[pl.ds(b + 1, 16)]
        cnt = (v1 - v0).astype(jnp.float32)
        inv_v[pl.ds(b, 16)] = 1.0 / jnp.maximum(cnt, 1.0)
        return u

    lax.fori_loop(jnp.int32(0), jnp.int32(BPW // 16), inv_grp, jnp.int32(0))

    pltpu.sync_copy(acc.at[pl.ds(0, BPW)], out_v)

    def fin_bag(b, u):
        iv = jnp.full((16,), inv_v[pl.ds(b, 16)][0], jnp.float32)
        for k in range(NV):
            out_v[b, k * 16:(k + 1) * 16] = jnp.maximum(
                out_v[b, k * 16:(k + 1) * 16] * iv
                + b1_v[k * 16:(k + 1) * 16], 0.0)
        return u

    lax.fori_loop(jnp.int32(0), jnp.int32(BPW), fin_bag, jnp.int32(0))
    pltpu.sync_copy(out_v, out.at[pl.ds(b0, BPW)])


def _bag(tab, xt_pad, xo_ext, b1, B):
    BPW = B // NW
    mesh = plsc.VectorSubcoreMesh(core_axis_name="c", subcore_axis_name="s",
                                  num_cores=NC, num_subcores=NS)
    f = pl.kernel(
        _bag_body,
        out_type=jax.ShapeDtypeStruct((B, DP), jnp.float32),
        mesh=mesh,
        compiler_params=pltpu.CompilerParams(needs_layout_passes=False,
                                             use_tc_tiling_on_sc=False),
        scratch_types=[
            pltpu.VMEM((BPW + 24,), jnp.int32),
            pltpu.VMEM((C,), jnp.int32),
            pltpu.VMEM((C,), jnp.int32),
            pltpu.VMEM((C, DP), jnp.bfloat16),
            pltpu.VMEM((C, DP), jnp.bfloat16),
            pltpu.VMEM((C,), jnp.int32),
            pltpu.VMEM((C,), jnp.int32),
            pltpu.VMEM((DP,), jnp.float32),
            pltpu.VMEM_SHARED((NS, BPW + 8, DP), jnp.float32),
            pltpu.VMEM((BPW, DP), jnp.float32),
            pltpu.VMEM((BPW + 16,), jnp.float32),
            pltpu.SemaphoreType.DMA,
            pltpu.SemaphoreType.DMA,
            pltpu.SemaphoreType.DMA,
            pltpu.SemaphoreType.DMA,
        ],
    )
    return f(tab, xt_pad, xo_ext, b1)


def _tail_kernel(h_ref, xn_ref, W2_ref, b2_ref, Wn1_ref, bn1_ref,
                 Wn2_ref, bn2_ref, Wc1_ref, bc1_ref, Wc2_ref, bc2_ref,
                 out_ref):
    f32 = jnp.float32
    h2 = jnp.maximum(jnp.dot(h_ref[...], W2_ref[...],
                             preferred_element_type=f32) + b2_ref[...], 0.0)
    n1 = jnp.maximum(jnp.dot(xn_ref[...], Wn1_ref[...],
                             preferred_element_type=f32) + bn1_ref[...], 0.0)
    n2 = jnp.maximum(jnp.dot(n1, Wn2_ref[...],
                             preferred_element_type=f32) + bn2_ref[...], 0.0)
    c1 = jnp.maximum(
        jnp.dot(h2, Wc1_ref[0:16, :], preferred_element_type=f32)
        + jnp.dot(n2, Wc1_ref[16:32, :], preferred_element_type=f32)
        + bc1_ref[...], 0.0)
    z = jnp.dot(c1, Wc2_ref[...], preferred_element_type=f32) + bc2_ref[...]
    out_ref[...] = jax.nn.sigmoid(z)


def _tail(h1, xn, W2, b2, Wn1, bn1, Wn2, bn2, Wc1, bc1, Wc2, bc2):
    B = h1.shape[0]
    BB = 2048
    full = lambda shape: pl.BlockSpec(shape, lambda i: tuple(0 for _ in shape))
    return pl.pallas_call(
        _tail_kernel,
        grid=(B // BB,),
        in_specs=[
            pl.BlockSpec((BB, h1.shape[1]), lambda i: (i, 0)),
            pl.BlockSpec((BB, xn.shape[1]), lambda i: (i, 0)),
            full(W2.shape), full(b2.shape), full(Wn1.shape), full(bn1.shape),
            full(Wn2.shape), full(bn2.shape), full(Wc1.shape), full(bc1.shape),
            full(Wc2.shape), full(bc2.shape),
        ],
        out_specs=pl.BlockSpec((BB, 1), lambda i: (i, 0)),
        out_shape=jax.ShapeDtypeStruct((B, 1), jnp.float32),
    )(h1, xn, W2, b2, Wn1, bn1, Wn2, bn2, Wc1, bc1, Wc2, bc2)


def kernel(xt, xo, xn, emb, W1, b1, W2, b2, Wn1, bn1, Wn2, bn2,
           Wc1, bc1, Wc2, bc2):
    T = xt.shape[0]
    B = xo.shape[0]
    tab = _proj(emb, W1)
    xt_pad = jnp.concatenate([xt.astype(jnp.int32),
                              jnp.zeros((C,), jnp.int32)])
    xo_ext = jnp.concatenate([xo.astype(jnp.int32),
                              jnp.full((24,), T, jnp.int32)])
    h1 = _bag(tab, xt_pad, xo_ext, b1, B)
    return _tail(h1, xn, W2, b2.reshape(1, -1), Wn1, bn1.reshape(1, -1),
                 Wn2, bn2.reshape(1, -1), Wc1, bc1.reshape(1, -1),
                 Wc2, bc2.reshape(1, -1))


# speedup vs baseline: 1.0653x; 1.0653x over previous
import functools

import jax
import jax.numpy as jnp
from jax import lax
from jax.experimental import pallas as pl
from jax.experimental.pallas import tpu as pltpu
from jax.experimental.pallas import tpu_sc as plsc

NC = 2
NS = 16
NW = NC * NS
C = 128
DP = 64
NV = DP // 16


def _proj_kernel(emb_ref, w_ref, out_ref):
    out_ref[...] = jnp.dot(emb_ref[...].astype(jnp.bfloat16),
                           w_ref[...].astype(jnp.bfloat16),
                           preferred_element_type=jnp.float32)


def _proj(emb, W1):
    V, D = emb.shape
    Dout = W1.shape[1]
    BS = 10000
    return pl.pallas_call(
        _proj_kernel,
        grid=(V // BS,),
        in_specs=[
            pl.BlockSpec((BS, D), lambda i: (i, 0)),
            pl.BlockSpec((D, Dout), lambda i: (0, 0)),
        ],
        out_specs=pl.BlockSpec((BS, Dout), lambda i: (i, 0)),
        out_shape=jax.ShapeDtypeStruct((V, Dout), jnp.float32),
    )(emb, W1)


def _scatter_add_rows(src_ref, acc_ref, seg_ref):
    pltpu.sync_copy(src_ref, acc_ref.at[seg_ref], add=True)


def _tile_slice(acc_sh, sid):
    return acc_sh.at[sid]


def _bag_body(tab, xt, xo, b1, out,
              xo_v, idx_0, idx_1, idx_2, idx_3, buf_0, buf_1, buf_2, buf_3,
              seg_a, seg_b, b1_v, acc_sh, out_v, inv_v,
              isem_0, isem_1, isem_2, isem_3,
              gsem_0, gsem_1, gsem_2, gsem_3):
    idx_r = (idx_0, idx_1, idx_2, idx_3)
    buf_r = (buf_0, buf_1, buf_2, buf_3)
    seg_r = (seg_a, seg_b)
    isem_r = (isem_0, isem_1, isem_2, isem_3)
    gsem_r = (gsem_0, gsem_1, gsem_2, gsem_3)
    BPW = out_v.shape[0]
    sid = lax.axis_index("s")
    wid = lax.axis_index("c") * NS + sid
    acc = _tile_slice(acc_sh, sid)
    b0 = pl.multiple_of(wid * BPW, BPW)
    pltpu.sync_copy(xo.at[pl.ds(b0, BPW + 24)], xo_v)
    pltpu.sync_copy(b1, b1_v)
    hdr = xo_v[pl.ds(0, 16)]
    s0 = hdr[0]
    e0 = xo_v[pl.ds(BPW, 16)][0]
    base = pl.multiple_of(jnp.bitwise_and(s0, jnp.int32(-8)), 8)
    nchunks = jnp.maximum(lax.div(e0 - base + (C - 1), jnp.int32(C)),
                          jnp.int32(1))

    zf = jnp.zeros((16,), jnp.float32)
    zi = jnp.zeros((16,), jnp.int32)
    ones = jnp.ones((16,), jnp.int32)
    lane = lax.iota(jnp.int32, 16)
    dummy = jnp.full((16,), jnp.int32(BPW), jnp.int32)

    def zero_out(r, u):
        for k in range(NV):
            out_v[r, k * 16:(k + 1) * 16] = zf
        return u

    lax.fori_loop(jnp.int32(0), jnp.int32(BPW), zero_out, jnp.int32(0))
    pltpu.sync_copy(out_v, acc.at[pl.ds(0, BPW)])

    def idx_start(c, idx_v, isem):
        off = pl.multiple_of(base + c * C, 8)
        pltpu.async_copy(xt.at[pl.ds(off, C)], idx_v, isem)

    def idx_wait(c, idx_v, isem):
        off = pl.multiple_of(base + c * C, 8)
        pltpu.make_async_copy(xt.at[pl.ds(off, C)], idx_v, isem).wait()

    def g_start(idx_v, buf_v, gsem):
        pltpu.async_copy(tab.at[idx_v], buf_v, gsem)

    def g_wait(idx_v, buf_v, gsem):
        pltpu.make_async_copy(tab.at[idx_v], buf_v, gsem).wait()

    idx_start(jnp.int32(0), idx_r[0], isem_r[0])

    @pl.when(nchunks > 1)
    def _():
        idx_start(jnp.int32(1), idx_r[1], isem_r[1])

    @pl.when(nchunks > 2)
    def _():
        idx_start(jnp.int32(2), idx_r[2], isem_r[2])

    idx_wait(jnp.int32(0), idx_r[0], isem_r[0])
    g_start(idx_r[0], buf_r[0], gsem_r[0])

    @pl.when(nchunks > 1)
    def _():
        idx_wait(jnp.int32(1), idx_r[1], isem_r[1])
        g_start(idx_r[1], buf_r[1], gsem_r[1])

    def half(c, sl, cur_b):
        idx_v, buf_v, seg_v = idx_r[sl], buf_r[sl], seg_r[sl % 2]
        s3, s2 = (sl + 3) % 4, (sl + 2) % 4

        @pl.when(c < nchunks)
        def _():
            g_wait(idx_v, buf_v, gsem_r[sl])

        @pl.when(c + 3 < nchunks)
        def _():
            idx_start(c + 3, idx_r[s3], isem_r[s3])

        @pl.when(c + 2 < nchunks)
        def _():
            idx_wait(c + 2, idx_r[s2], isem_r[s2])
            g_start(idx_r[s2], buf_r[s2], gsem_r[s2])

        lo = base + c * C
        hi = jnp.minimum(lo + C, e0)

        def cnt_body(k, cv):
            v = xo_v[pl.ds(k * 16 + 1, 16)]
            return cv + plsc.all_reduce_population_count(v <= hi)[0]

        total = lax.fori_loop(jnp.int32(0), jnp.int32(BPW // 16), cnt_body,
                              jnp.int32(0))

        @pl.when(c < nchunks)
        def _():
            for g in range(C // 16):
                seg_v[g * 16:(g + 1) * 16] = zi
            nstarts = total - cur_b
            ngroups = lax.div(nstarts + jnp.int32(15), jnp.int32(16))

            def inc_body(gi, u):
                off = cur_b + 1 + gi * 16
                v = xo_v[pl.ds(off, 16)]
                m = jnp.logical_and(lane < (nstarts - gi * 16), v < hi)
                plsc.addupdate_scatter(seg_v, [v - lo], ones, mask=m)
                return u

            lax.fori_loop(jnp.int32(0), ngroups, inc_body, jnp.int32(0))

            s0ml = s0 - lo
            e0ml = e0 - lo
            carry = cur_b
            for g in range(C // 16):
                inc = seg_v[g * 16:(g + 1) * 16]
                cs = plsc.cumsum(inc) + carry
                carry = cs[15]
                pos = lane + g * 16
                valid = jnp.logical_and(pos >= s0ml, pos < e0ml)
                seg_v[g * 16:(g + 1) * 16] = jnp.where(valid, cs, dummy)

            _scatter_add_rows(buf_v, acc, seg_v)

        return total

    def quad_body(p, cur_b):
        c = p * 4
        for sl in range(4):
            cur_b = half(c + sl, sl, cur_b)
        return cur_b

    nquads = lax.div(nchunks + 3, jnp.int32(4))
    lax.fori_loop(jnp.int32(0), nquads, quad_body, jnp.int32(0))

    def inv_grp(g, u):
        b = g * 16
        v0 = xo_v[pl.ds(b, 16)]
        v1 = xo_v[pl.ds(b + 1, 16)]
        cnt = (v1 - v0).astype(jnp.float32)
        inv_v[pl.ds(b, 16)] = 1.0 / jnp.maximum(cnt, 1.0)
        return u

    lax.fori_loop(jnp.int32(0), jnp.int32(BPW // 16), inv_grp, jnp.int32(0))

    pltpu.sync_copy(acc.at[pl.ds(0, BPW)], out_v)

    def fin_bag(b, u):
        iv = jnp.full((16,), inv_v[pl.ds(b, 16)][0], jnp.float32)
        for k in range(NV):
            out_v[b, k * 16:(k + 1) * 16] = jnp.maximum(
                out_v[b, k * 16:(k + 1) * 16] * iv
                + b1_v[k * 16:(k + 1) * 16], 0.0)
        return u

    lax.fori_loop(jnp.int32(0), jnp.int32(BPW), fin_bag, jnp.int32(0))
    pltpu.sync_copy(out_v, out.at[pl.ds(b0, BPW)])


def _bag(tab, xt_pad, xo_ext, b1, B):
    BPW = B // NW
    mesh = plsc.VectorSubcoreMesh(core_axis_name="c", subcore_axis_name="s",
                                  num_cores=NC, num_subcores=NS)
    f = pl.kernel(
        _bag_body,
        out_type=jax.ShapeDtypeStruct((B, DP), jnp.float32),
        mesh=mesh,
        compiler_params=pltpu.CompilerParams(needs_layout_passes=False,
                                             use_tc_tiling_on_sc=False),
        scratch_types=(
            [pltpu.VMEM((BPW + 24,), jnp.int32)]
            + [pltpu.VMEM((C,), jnp.int32) for _ in range(4)]
            + [pltpu.VMEM((C, DP), jnp.float32) for _ in range(4)]
            + [pltpu.VMEM((C,), jnp.int32) for _ in range(2)]
            + [pltpu.VMEM((DP,), jnp.float32),
               pltpu.VMEM_SHARED((NS, BPW + 8, DP), jnp.float32),
               pltpu.VMEM((BPW, DP), jnp.float32),
               pltpu.VMEM((BPW + 16,), jnp.float32)]
            + [pltpu.SemaphoreType.DMA for _ in range(8)]
        ),
    )
    return f(tab, xt_pad, xo_ext, b1)


def _tail_kernel(h_ref, xn_ref, W2_ref, b2_ref, Wn1_ref, bn1_ref,
                 Wn2_ref, bn2_ref, Wc1_ref, bc1_ref, Wc2_ref, bc2_ref,
                 out_ref):
    f32 = jnp.float32
    h2 = jnp.maximum(jnp.dot(h_ref[...], W2_ref[...],
                             preferred_element_type=f32) + b2_ref[...], 0.0)
    n1 = jnp.maximum(jnp.dot(xn_ref[...], Wn1_ref[...],
                             preferred_element_type=f32) + bn1_ref[...], 0.0)
    n2 = jnp.maximum(jnp.dot(n1, Wn2_ref[...],
                             preferred_element_type=f32) + bn2_ref[...], 0.0)
    c1 = jnp.maximum(
        jnp.dot(h2, Wc1_ref[0:16, :], preferred_element_type=f32)
        + jnp.dot(n2, Wc1_ref[16:32, :], preferred_element_type=f32)
        + bc1_ref[...], 0.0)
    z = jnp.dot(c1, Wc2_ref[...], preferred_element_type=f32) + bc2_ref[...]
    out_ref[...] = jax.nn.sigmoid(z)


def _tail(h1, xn, W2, b2, Wn1, bn1, Wn2, bn2, Wc1, bc1, Wc2, bc2):
    B = h1.shape[0]
    BB = 2048
    full = lambda shape: pl.BlockSpec(shape, lambda i: tuple(0 for _ in shape))
    return pl.pallas_call(
        _tail_kernel,
        grid=(B // BB,),
        in_specs=[
            pl.BlockSpec((BB, h1.shape[1]), lambda i: (i, 0)),
            pl.BlockSpec((BB, xn.shape[1]), lambda i: (i, 0)),
            full(W2.shape), full(b2.shape), full(Wn1.shape), full(bn1.shape),
            full(Wn2.shape), full(bn2.shape), full(Wc1.shape), full(bc1.shape),
            full(Wc2.shape), full(bc2.shape),
        ],
        out_specs=pl.BlockSpec((BB, 1), lambda i: (i, 0)),
        out_shape=jax.ShapeDtypeStruct((B, 1), jnp.float32),
    )(h1, xn, W2, b2, Wn1, bn1, Wn2, bn2, Wc1, bc1, Wc2, bc2)


def kernel(xt, xo, xn, emb, W1, b1, W2, b2, Wn1, bn1, Wn2, bn2,
           Wc1, bc1, Wc2, bc2):
    T = xt.shape[0]
    B = xo.shape[0]
    tab = _proj(emb, W1)
    xt_pad = jnp.concatenate([xt.astype(jnp.int32),
                              jnp.zeros((C,), jnp.int32)])
    xo_ext = jnp.concatenate([xo.astype(jnp.int32),
                              jnp.full((24,), T, jnp.int32)])
    h1 = _bag(tab, xt_pad, xo_ext, b1, B)
    return _tail(h1, xn, W2, b2.reshape(1, -1), Wn1, bn1.reshape(1, -1),
                 Wn2, bn2.reshape(1, -1), Wc1, bc1.reshape(1, -1),
                 Wc2, bc2.reshape(1, -1))
